# baseline (speedup 1.0000x reference)
import functools

import jax
import jax.numpy as jnp
from jax import lax
from jax.experimental import pallas as pl
from jax.experimental.pallas import tpu as pltpu
from jax.experimental.pallas import tpu_sc as plsc

_GW = 128


@functools.lru_cache(maxsize=None)
def _build_format(vocab, d):
    info = plsc.get_sparse_core_info()
    nw = info.num_cores * info.num_subcores
    nvb = vocab // 128
    per_w = nvb // nw
    n_extra = nvb - per_w * nw
    tail = vocab - nvb * 128
    assert per_w % 2 == 0

    mesh = plsc.VectorSubcoreMesh(core_axis_name="c", subcore_axis_name="s")

    @functools.partial(
        pl.kernel,
        mesh=mesh,
        out_type=jax.ShapeDtypeStruct((vocab, 128), jnp.float32),
        scratch_types=[
            pltpu.VMEM((d, 128), jnp.float32),
            pltpu.VMEM((d, 128), jnp.float32),
            pltpu.VMEM((128, 128), jnp.float32),
            pltpu.VMEM((128, 128), jnp.float32),
            pltpu.SemaphoreType.DMA,
            pltpu.SemaphoreType.DMA,
            pltpu.SemaphoreType.DMA,
            pltpu.SemaphoreType.DMA,
        ],
        compiler_params=pltpu.CompilerParams(needs_layout_passes=False),
    )
    def fmt(tabT_hbm, tail_hbm, out_hbm, in0, in1, t0, t1, sl0, sl1, sw0, sw1):
        wid = lax.axis_index("s") * info.num_cores + lax.axis_index("c")
        base = wid * per_w
        bufs = ((in0, t0, sl0, sw0), (in1, t1, sl1, sw1))
        lanes = lax.iota(jnp.int32, 16)

        def dst_slc(vb):
            return out_hbm.at[pl.ds(vb * 128, 128), :]

        def start_load(vb, in_v, sl):
            for db in range(d // 8):
                pltpu.async_copy(
                    tabT_hbm.at[pl.ds(db * 8, 8), pl.ds(vb * 128, 128)],
                    in_v.at[pl.ds(db * 8, 8), :], sl)

        def wait_load(vb, in_v, sl):
            for db in range(d // 8):
                pltpu.make_async_copy(
                    tabT_hbm.at[pl.ds(db * 8, 8), pl.ds(vb * 128, 128)],
                    in_v.at[pl.ds(db * 8, 8), :], sl).wait()

        def start_write(vb, t_v, sw):
            pltpu.async_copy(t_v, dst_slc(vb), sw)

        def wait_write(vb, t_v, sw):
            pltpu.make_async_copy(t_v, dst_slc(vb), sw).wait()

        rows_q = tuple(lanes + q * 16 for q in range(d // 16))

        def transpose(in_v, t_v, nq):
            unroll = 8

            def blk(i, cols0):
                v0 = i * unroll
                vals = [
                    plsc.load_gather(in_v, [rows_q[q], cols0 + u])
                    for u in range(unroll)
                    for q in range(d // 16)
                ]
                for u in range(unroll):
                    for q in range(d // 16):
                        t_v[v0 + u, pl.ds(q * 16, 16)] = \
                            vals[u * (d // 16) + q]
                return cols0 + unroll

            lax.fori_loop(0, 16 * nq // unroll, blk,
                          jnp.zeros((16,), jnp.int32))

        def do_chunk(g, in_v, t_v, sl, sw, first, last):
            vb = base + g
            wait_load(vb, in_v, sl)
            if not first:
                wait_write(vb - 2, t_v, sw)
            transpose(in_v, t_v, 8)
            start_write(vb, t_v, sw)
            if not last:
                start_load(vb + 2, in_v, sl)

        for bi, (in_v, t_v, sl, sw) in enumerate(bufs):
            start_load(base + bi, in_v, sl)
        for bi, (in_v, t_v, sl, sw) in enumerate(bufs):
            do_chunk(bi, in_v, t_v, sl, sw, True, False)

        def body(i, carry):
            for bi, (in_v, t_v, sl, sw) in enumerate(bufs):
                do_chunk(2 * i + bi, in_v, t_v, sl, sw, False, False)
            return carry

        lax.fori_loop(1, per_w // 2 - 1, body, 0)

        for bi, (in_v, t_v, sl, sw) in enumerate(bufs):
            do_chunk(per_w - 2 + bi, in_v, t_v, sl, sw, False, True)
        for bi, (in_v, t_v, sl, sw) in enumerate(bufs):
            wait_write(base + per_w - 2 + bi, t_v, sw)

        @pl.when(wid < n_extra)
        def _extra():
            vb = nw * per_w + wid
            for db in range(d // 8):
                pltpu.sync_copy(
                    tabT_hbm.at[pl.ds(db * 8, 8), pl.ds(vb * 128, 128)],
                    in0.at[pl.ds(db * 8, 8), :])
            transpose(in0, t0, 8)
            pltpu.sync_copy(t0, dst_slc(vb))

        if tail:
            @pl.when(wid == nw - 1)
            def _tail():
                pltpu.sync_copy(tail_hbm, in0)
                pltpu.sync_copy(in0, out_hbm.at[pl.ds(nvb * 128, tail), :])

    return fmt


@functools.lru_cache(maxsize=None)
def _build_gather(n, vocab, dp):
    info = plsc.get_sparse_core_info()
    nw = info.num_cores * info.num_subcores
    bpw = n // nw
    k = 2
    chunk = k * _GW
    n_chunks = bpw // chunk
    assert n % nw == 0 and bpw % chunk == 0 and n_chunks % 2 == 0

    mesh = plsc.VectorSubcoreMesh(core_axis_name="c", subcore_axis_name="s")

    @functools.partial(
        pl.kernel,
        mesh=mesh,
        out_type=jax.ShapeDtypeStruct((n, dp), jnp.float32),
        scratch_types=[
            pltpu.VMEM((k, _GW), jnp.int32),
            pltpu.VMEM((k, _GW), jnp.int32),
            pltpu.VMEM((chunk, dp), jnp.float32),
            pltpu.VMEM((chunk, dp), jnp.float32),
            pltpu.SemaphoreType.DMA,
            pltpu.SemaphoreType.DMA,
            pltpu.SemaphoreType.DMA,
            pltpu.SemaphoreType.DMA,
        ],
    )
    def gather(idx_hbm, table_hbm, out_hbm, idx0, idx1, rows0, rows1,
               sg0, sg1, sw0, sw1):
        wid = lax.axis_index("s") * info.num_cores + lax.axis_index("c")
        base = wid * bpw
        bufs = ((idx0, rows0, sg0, sw0), (idx1, rows1, sg1, sw1))

        def start_gather(g, idx_v, rows_v, sg):
            for j in range(k):
                pltpu.sync_copy(
                    idx_hbm.at[pl.ds(base + g * chunk + j * _GW, _GW)],
                    idx_v.at[j])
            for j in range(k):
                pltpu.async_copy(table_hbm.at[idx_v.at[j]],
                                 rows_v.at[pl.ds(j * _GW, _GW), :], sg)

        def wait_gather(idx_v, rows_v, sg):
            for j in range(k):
                pltpu.make_async_copy(table_hbm.at[idx_v.at[j]],
                                      rows_v.at[pl.ds(j * _GW, _GW), :],
                                      sg).wait()

        def start_wb(g, rows_v, sw):
            pltpu.async_copy(rows_v,
                             out_hbm.at[pl.ds(base + g * chunk, chunk), :], sw)

        def wait_wb(g, rows_v, sw):
            pltpu.make_async_copy(
                rows_v, out_hbm.at[pl.ds(base + g * chunk, chunk), :],
                sw).wait()

        for bi, (idx_v, rows_v, sg, sw) in enumerate(bufs):
            start_gather(bi, idx_v, rows_v, sg)

        def body(i, carry):
            g0 = 2 * i
            for bi, (idx_v, rows_v, sg, sw) in enumerate(bufs):
                wait_gather(idx_v, rows_v, sg)
                start_wb(g0 + bi, rows_v, sw)
            for bi, (idx_v, rows_v, sg, sw) in enumerate(bufs):
                wait_wb(g0 + bi, rows_v, sw)
                start_gather(g0 + bi + 2, idx_v, rows_v, sg)
            return carry

        lax.fori_loop(0, n_chunks // 2 - 1, body, 0)

        gl = n_chunks - 2
        for bi, (idx_v, rows_v, sg, sw) in enumerate(bufs):
            wait_gather(idx_v, rows_v, sg)
            start_wb(gl + bi, rows_v, sw)
        for bi, (idx_v, rows_v, sg, sw) in enumerate(bufs):
            wait_wb(gl + bi, rows_v, sw)

    return gather


def kernel(x, table):
    b, l = x.shape
    vocab, d = table.shape
    idx = x.astype(jnp.int32).reshape(-1)
    tabT = jnp.swapaxes(table, 0, 1)
    tail = vocab - (vocab // 128) * 128
    tail_rows = jnp.pad(table[vocab - tail:], ((0, 0), (0, 128 - d)))
    tpad = _build_format(vocab, d)(tabT, tail_rows)
    out = _build_gather(b * l, vocab, 128)(idx, tpad)
    return out.reshape(b, l, 128)[:, :, :d]


# speedup vs baseline: 1.5709x; 1.5709x over previous
import functools

import jax
import jax.numpy as jnp
from jax import lax
from jax.experimental import pallas as pl
from jax.experimental.pallas import tpu as pltpu
from jax.experimental.pallas import tpu_sc as plsc

_GW = 128


@functools.lru_cache(maxsize=None)
def _build_gather(n, vocab, dp):
    info = plsc.get_sparse_core_info()
    nw = info.num_cores * info.num_subcores
    bpw = n // nw
    k = 2
    chunk = k * _GW
    n_chunks = bpw // chunk
    assert n % nw == 0 and bpw % chunk == 0 and n_chunks % 2 == 0

    mesh = plsc.VectorSubcoreMesh(core_axis_name="c", subcore_axis_name="s")

    @functools.partial(
        pl.kernel,
        mesh=mesh,
        out_type=jax.ShapeDtypeStruct((n, dp), jnp.float32),
        scratch_types=[
            pltpu.VMEM((k, _GW), jnp.int32),
            pltpu.VMEM((k, _GW), jnp.int32),
            pltpu.VMEM((chunk, dp), jnp.float32),
            pltpu.VMEM((chunk, dp), jnp.float32),
            pltpu.SemaphoreType.DMA,
            pltpu.SemaphoreType.DMA,
            pltpu.SemaphoreType.DMA,
            pltpu.SemaphoreType.DMA,
        ],
    )
    def gather(idx_hbm, table_hbm, out_hbm, idx0, idx1, rows0, rows1,
               sg0, sg1, sw0, sw1):
        wid = lax.axis_index("s") * info.num_cores + lax.axis_index("c")
        base = wid * bpw
        bufs = ((idx0, rows0, sg0, sw0), (idx1, rows1, sg1, sw1))

        def start_gather(g, idx_v, rows_v, sg):
            for j in range(k):
                pltpu.sync_copy(
                    idx_hbm.at[pl.ds(base + g * chunk + j * _GW, _GW)],
                    idx_v.at[j])
            for j in range(k):
                pltpu.async_copy(table_hbm.at[idx_v.at[j]],
                                 rows_v.at[pl.ds(j * _GW, _GW), :], sg)

        def wait_gather(idx_v, rows_v, sg):
            for j in range(k):
                pltpu.make_async_copy(table_hbm.at[idx_v.at[j]],
                                      rows_v.at[pl.ds(j * _GW, _GW), :],
                                      sg).wait()

        def start_wb(g, rows_v, sw):
            pltpu.async_copy(rows_v,
                             out_hbm.at[pl.ds(base + g * chunk, chunk), :], sw)

        def wait_wb(g, rows_v, sw):
            pltpu.make_async_copy(
                rows_v, out_hbm.at[pl.ds(base + g * chunk, chunk), :],
                sw).wait()

        for bi, (idx_v, rows_v, sg, sw) in enumerate(bufs):
            start_gather(bi, idx_v, rows_v, sg)

        def body(i, carry):
            g0 = 2 * i
            for bi, (idx_v, rows_v, sg, sw) in enumerate(bufs):
                wait_gather(idx_v, rows_v, sg)
                start_wb(g0 + bi, rows_v, sw)
            for bi, (idx_v, rows_v, sg, sw) in enumerate(bufs):
                wait_wb(g0 + bi, rows_v, sw)
                start_gather(g0 + bi + 2, idx_v, rows_v, sg)
            return carry

        lax.fori_loop(0, n_chunks // 2 - 1, body, 0)

        gl = n_chunks - 2
        for bi, (idx_v, rows_v, sg, sw) in enumerate(bufs):
            wait_gather(idx_v, rows_v, sg)
            start_wb(gl + bi, rows_v, sw)
        for bi, (idx_v, rows_v, sg, sw) in enumerate(bufs):
            wait_wb(gl + bi, rows_v, sw)

    return gather


def kernel(x, table):
    b, l = x.shape
    vocab, d = table.shape
    idx = x.astype(jnp.int32).reshape(-1)
    tpad = jnp.pad(table, ((0, 0), (0, 128 - d)))
    out = _build_gather(b * l, vocab, 128)(idx, tpad)
    return out.reshape(b, l, 128)[:, :, :d]
